# baseline (speedup 1.0000x reference)
import jax
import jax.numpy as jnp
from jax.experimental import pallas as pl

SEQ = 1024
DIM = 128
KEEP = int(0.3 * SEQ * SEQ)

_HI = jax.lax.Precision.HIGHEST
_MININT = -(2**31)


def _gcn_kernel(x_ref, w_ref, b_ref, out_ref):
    xs = x_ref[...]

    ew = jax.lax.dot_general(
        xs, xs, (((1,), (1,)), ((), ())),
        preferred_element_type=jnp.float32)
    ii = jax.lax.broadcasted_iota(jnp.int32, (SEQ, SEQ), 0)
    jj = jax.lax.broadcasted_iota(jnp.int32, (SEQ, SEQ), 1)
    ew = ew - jnp.where(ii == jj, 1.0, 0.0).astype(jnp.float32)

    ibits = jax.lax.bitcast_convert_type(ew, jnp.int32)
    key = jnp.where(ibits >= 0, ibits, ibits ^ 0x7FFFFFFF)

    hi16 = jax.lax.shift_right_arithmetic(key, 16).astype(jnp.int16)

    def hi_step(i, ob):
        cand = ob | jnp.left_shift(1, 15 - i)
        cs = (cand ^ 0x8000).astype(jnp.int16)
        m = (hi16 >= cs).astype(jnp.int16)
        cnt = jnp.sum(jnp.sum(m, axis=0, keepdims=True).astype(jnp.int32))
        return jnp.where(cnt >= KEEP, cand, ob)

    ohi = jax.lax.fori_loop(0, 16, hi_step, jnp.int32(0))
    thi = (ohi ^ 0x8000).astype(jnp.int16)

    n_gt_hi = jnp.sum(jnp.sum(
        (hi16 > thi).astype(jnp.int16), axis=0, keepdims=True
    ).astype(jnp.int32))
    lo_off = key.astype(jnp.int16) ^ (-32768)
    lo_m = jnp.where(hi16 == thi, lo_off, -32768).astype(jnp.int16)

    def lo_step(i, ol):
        cand = ol | jnp.left_shift(1, 15 - i)
        cs = (cand ^ 0x8000).astype(jnp.int16)
        m = (lo_m >= cs).astype(jnp.int16)
        cnt = n_gt_hi + jnp.sum(jnp.sum(
            m, axis=0, keepdims=True).astype(jnp.int32))
        return jnp.where(cnt >= KEEP, cand, ol)

    olo = jax.lax.fori_loop(0, 16, lo_step, jnp.int32(0))
    t_key = (jnp.left_shift(ohi, 16) | olo) ^ _MININT

    above = key > t_key
    tie = key == t_key
    n_above = jnp.sum(above.astype(jnp.int32))
    r_f = (KEEP - n_above).astype(jnp.float32)

    tie_bf = jnp.where(tie, 1.0, 0.0).astype(jnp.bfloat16)
    l_bf = jnp.where(jj < ii, 1.0, 0.0).astype(jnp.bfloat16)
    wc = jax.lax.dot_general(
        l_bf, tie_bf, (((1,), (0,)), ((), ())),
        preferred_element_type=jnp.float32)
    rc = wc[SEQ - 1:SEQ, :] + tie[SEQ - 1:SEQ, :].astype(jnp.float32)
    inc = rc
    s = 1
    while s < SEQ:
        inc = inc + jnp.concatenate(
            [jnp.zeros((1, s), jnp.float32), inc[:, :SEQ - s]], axis=1)
        s *= 2
    row_off = inc - rc

    keep_t = above | (tie & ((row_off + wc) < r_f))
    at = jnp.where(keep_t, ew, 0.0)

    deg = jnp.sum(at, axis=1, keepdims=True) + 1.0
    dis = deg ** -0.5
    dis = jnp.where(jnp.isinf(dis), 0.0, dis)

    xt = jax.lax.dot_general(
        xs, w_ref[...], (((1,), (1,)), ((), ())),
        preferred_element_type=jnp.float32, precision=_HI)

    sx = dis * xt
    y = jax.lax.dot_general(
        at, sx, (((1,), (0,)), ((), ())),
        preferred_element_type=jnp.float32, precision=_HI)

    out_ref[...] = dis * y + (dis * dis) * xt + b_ref[...]


def kernel(x, W, b):
    xs = x.reshape(SEQ, DIM)
    b2 = b.reshape(1, DIM)
    out = pl.pallas_call(
        _gcn_kernel,
        out_shape=jax.ShapeDtypeStruct((SEQ, DIM), jnp.float32),
    )(xs, W, b2)
    return out[None, :, :]


# speedup vs baseline: 1.7180x; 1.7180x over previous
import jax
import jax.numpy as jnp
from jax.experimental import pallas as pl

SEQ = 1024
DIM = 128
KEEP = int(0.3 * SEQ * SEQ)

_HI = jax.lax.Precision.HIGHEST
_MININT = -(2**31)


def _gcn_kernel(x_ref, w_ref, b_ref, out_ref):
    xs = x_ref[...]

    ew = jax.lax.dot_general(
        xs, xs, (((1,), (1,)), ((), ())),
        preferred_element_type=jnp.float32)
    ii = jax.lax.broadcasted_iota(jnp.int32, (SEQ, SEQ), 0)
    jj = jax.lax.broadcasted_iota(jnp.int32, (SEQ, SEQ), 1)
    ew = ew - jnp.where(ii == jj, 1.0, 0.0).astype(jnp.float32)

    ibits = jax.lax.bitcast_convert_type(ew, jnp.int32)
    key = jnp.where(ibits >= 0, ibits, ibits ^ 0x7FFFFFFF)

    def count_ge(arr, cs):
        part = jnp.sum(jnp.where(arr >= cs, 1.0, 0.0), axis=0, keepdims=True)
        return jnp.sum(part)

    def value_step(i, obits):
        cand = obits | jnp.left_shift(1, 31 - i)
        cnt = count_ge(key, cand ^ _MININT)
        return jnp.where(cnt >= float(KEEP), cand, obits)

    obits = jax.lax.fori_loop(0, 32, value_step, jnp.int32(0))
    t_key = obits ^ _MININT

    above = key > t_key
    tie = key == t_key
    n_above = jnp.sum(above.astype(jnp.int32))
    r_f = (KEEP - n_above).astype(jnp.float32)

    tie_bf = jnp.where(tie, 1.0, 0.0).astype(jnp.bfloat16)
    l_bf = jnp.where(jj < ii, 1.0, 0.0).astype(jnp.bfloat16)
    wc = jax.lax.dot_general(
        l_bf, tie_bf, (((1,), (0,)), ((), ())),
        preferred_element_type=jnp.float32)
    rc = wc[SEQ - 1:SEQ, :] + tie[SEQ - 1:SEQ, :].astype(jnp.float32)
    inc = rc
    s = 1
    while s < SEQ:
        inc = inc + jnp.concatenate(
            [jnp.zeros((1, s), jnp.float32), inc[:, :SEQ - s]], axis=1)
        s *= 2
    row_off = inc - rc

    keep_t = above | (tie & ((row_off + wc) < r_f))
    at = jnp.where(keep_t, ew, 0.0)

    deg = jnp.sum(at, axis=1, keepdims=True) + 1.0
    dis = deg ** -0.5
    dis = jnp.where(jnp.isinf(dis), 0.0, dis)

    xt = jax.lax.dot_general(
        xs, w_ref[...], (((1,), (1,)), ((), ())),
        preferred_element_type=jnp.float32, precision=_HI)

    sx = dis * xt
    y = jax.lax.dot_general(
        at, sx, (((1,), (0,)), ((), ())),
        preferred_element_type=jnp.float32, precision=_HI)

    out_ref[...] = dis * y + (dis * dis) * xt + b_ref[...]


def kernel(x, W, b):
    xs = x.reshape(SEQ, DIM)
    b2 = b.reshape(1, DIM)
    out = pl.pallas_call(
        _gcn_kernel,
        out_shape=jax.ShapeDtypeStruct((SEQ, DIM), jnp.float32),
    )(xs, W, b2)
    return out[None, :, :]


# speedup vs baseline: 2.0606x; 1.1994x over previous
import jax
import jax.numpy as jnp
from jax.experimental import pallas as pl

SEQ = 1024
DIM = 128
KEEP = int(0.3 * SEQ * SEQ)

_HI = jax.lax.Precision.HIGHEST
_MININT = -(2**31)


def _gcn_kernel(x_ref, w_ref, b_ref, out_ref):
    xs = x_ref[...]

    ew = jax.lax.dot_general(
        xs, xs, (((1,), (1,)), ((), ())),
        preferred_element_type=jnp.float32)
    ii = jax.lax.broadcasted_iota(jnp.int32, (SEQ, SEQ), 0)
    jj = jax.lax.broadcasted_iota(jnp.int32, (SEQ, SEQ), 1)
    ew = ew - jnp.where(ii == jj, 1.0, 0.0).astype(jnp.float32)

    ibits = jax.lax.bitcast_convert_type(ew, jnp.int32)
    key = jnp.where(ibits >= 0, ibits, ibits ^ 0x7FFFFFFF)

    ud = jax.lax.broadcasted_iota(jnp.int32, (128, 128), 0)
    vd = jax.lax.broadcasted_iota(jnp.int32, (128, 128), 1)
    wdg = jnp.where(vd > ud, 2.0,
                    jnp.where(vd == ud, 1.0, 0.0)).astype(jnp.float32)

    def count_cmp(cs, strict):
        total = 0.0
        for blk in range(8):
            lo = 128 * blk
            dsub = key[lo:lo + 128, lo:lo + 128]
            md = (dsub > cs) if strict else (dsub >= cs)
            total += jnp.sum(
                jnp.sum(jnp.where(md, wdg, 0.0), axis=0, keepdims=True))
            if blk < 7:
                rsub = key[lo:lo + 128, lo + 128:]
                mr = (rsub > cs) if strict else (rsub >= cs)
                total += jnp.sum(
                    jnp.sum(jnp.where(mr, 2.0, 0.0), axis=0, keepdims=True))
        return total

    def value_step(i, obits):
        cand = obits | jnp.left_shift(1, 31 - i)
        cnt = count_cmp(cand ^ _MININT, False)
        return jnp.where(cnt >= float(KEEP), cand, obits)

    obits = jax.lax.fori_loop(0, 32, value_step, jnp.int32(0))
    t_key = obits ^ _MININT

    above = key > t_key
    tie = key == t_key
    r_f = float(KEEP) - count_cmp(t_key, True)

    tie_bf = jnp.where(tie, 1.0, 0.0).astype(jnp.bfloat16)
    l_bf = jnp.where(jj < ii, 1.0, 0.0).astype(jnp.bfloat16)
    wc = jax.lax.dot_general(
        l_bf, tie_bf, (((1,), (0,)), ((), ())),
        preferred_element_type=jnp.float32)
    rc = wc[SEQ - 1:SEQ, :] + tie[SEQ - 1:SEQ, :].astype(jnp.float32)
    inc = rc
    s = 1
    while s < SEQ:
        inc = inc + jnp.concatenate(
            [jnp.zeros((1, s), jnp.float32), inc[:, :SEQ - s]], axis=1)
        s *= 2
    row_off = inc - rc

    keep_t = above | (tie & ((row_off + wc) < r_f))
    at = jnp.where(keep_t, ew, 0.0)

    deg = jnp.sum(at, axis=1, keepdims=True) + 1.0
    dis = deg ** -0.5
    dis = jnp.where(jnp.isinf(dis), 0.0, dis)

    xt = jax.lax.dot_general(
        xs, w_ref[...], (((1,), (1,)), ((), ())),
        preferred_element_type=jnp.float32, precision=_HI)

    sx = dis * xt
    y = jax.lax.dot_general(
        at, sx, (((1,), (0,)), ((), ())),
        preferred_element_type=jnp.float32, precision=_HI)

    out_ref[...] = dis * y + (dis * dis) * xt + b_ref[...]


def kernel(x, W, b):
    xs = x.reshape(SEQ, DIM)
    b2 = b.reshape(1, DIM)
    out = pl.pallas_call(
        _gcn_kernel,
        out_shape=jax.ShapeDtypeStruct((SEQ, DIM), jnp.float32),
    )(xs, W, b2)
    return out[None, :, :]


# speedup vs baseline: 2.6822x; 1.3017x over previous
import jax
import jax.numpy as jnp
from jax.experimental import pallas as pl

SEQ = 1024
DIM = 128
KEEP = int(0.3 * SEQ * SEQ)

_MININT = -(2**31)


def _gcn_kernel(x_ref, w_ref, b_ref, out_ref):
    xs = x_ref[...]

    ew = jax.lax.dot_general(
        xs, xs, (((1,), (1,)), ((), ())),
        preferred_element_type=jnp.float32)
    ii = jax.lax.broadcasted_iota(jnp.int32, (SEQ, SEQ), 0)
    jj = jax.lax.broadcasted_iota(jnp.int32, (SEQ, SEQ), 1)
    ew = ew - jnp.where(ii == jj, 1.0, 0.0).astype(jnp.float32)

    ibits = jax.lax.bitcast_convert_type(ew, jnp.int32)
    key = jnp.where(ibits >= 0, ibits, ibits ^ 0x7FFFFFFF)

    ud = jax.lax.broadcasted_iota(jnp.int32, (128, 128), 0)
    vd = jax.lax.broadcasted_iota(jnp.int32, (128, 128), 1)
    wdg = jnp.where(vd > ud, 2.0,
                    jnp.where(vd == ud, 1.0, 0.0)).astype(jnp.float32)

    def count_cmp(cs, strict):
        total = 0.0
        for blk in range(8):
            lo = 128 * blk
            dsub = key[lo:lo + 128, lo:lo + 128]
            md = (dsub > cs) if strict else (dsub >= cs)
            total += jnp.sum(
                jnp.sum(jnp.where(md, wdg, 0.0), axis=0, keepdims=True))
            if blk < 7:
                rsub = key[lo:lo + 128, lo + 128:]
                mr = (rsub > cs) if strict else (rsub >= cs)
                total += jnp.sum(
                    jnp.sum(jnp.where(mr, 2.0, 0.0), axis=0, keepdims=True))
        return total

    def value_step(i, obits):
        cand = obits | jnp.left_shift(1, 31 - i)
        cnt = count_cmp(cand ^ _MININT, False)
        return jnp.where(cnt >= float(KEEP), cand, obits)

    obits = jax.lax.fori_loop(0, 32, value_step, jnp.int32(0))
    t_key = obits ^ _MININT

    above = key > t_key
    tie = key == t_key
    r_f = float(KEEP) - count_cmp(t_key, True)

    tie_bf = jnp.where(tie, 1.0, 0.0).astype(jnp.bfloat16)
    l_bf = jnp.where(jj < ii, 1.0, 0.0).astype(jnp.bfloat16)
    wc = jax.lax.dot_general(
        l_bf, tie_bf, (((1,), (0,)), ((), ())),
        preferred_element_type=jnp.float32)
    rc = wc[SEQ - 1:SEQ, :] + tie[SEQ - 1:SEQ, :].astype(jnp.float32)
    inc = rc
    s = 1
    while s < SEQ:
        inc = inc + jnp.concatenate(
            [jnp.zeros((1, s), jnp.float32), inc[:, :SEQ - s]], axis=1)
        s *= 2
    row_off = inc - rc

    keep_t = above | (tie & ((row_off + wc) < r_f))
    at = jnp.where(keep_t, ew, 0.0)

    deg = jnp.sum(at, axis=1, keepdims=True) + 1.0
    dis = deg ** -0.5
    dis = jnp.where(jnp.isinf(dis), 0.0, dis)

    xt = jax.lax.dot_general(
        xs, w_ref[...], (((1,), (1,)), ((), ())),
        preferred_element_type=jnp.float32)

    sx = dis * xt
    y = jax.lax.dot_general(
        at, sx, (((1,), (0,)), ((), ())),
        preferred_element_type=jnp.float32)

    out_ref[...] = dis * y + (dis * dis) * xt + b_ref[...]


def kernel(x, W, b):
    xs = x.reshape(SEQ, DIM)
    b2 = b.reshape(1, DIM)
    out = pl.pallas_call(
        _gcn_kernel,
        out_shape=jax.ShapeDtypeStruct((SEQ, DIM), jnp.float32),
    )(xs, W, b2)
    return out[None, :, :]
